# baseline (speedup 1.0000x reference)
import functools

import jax
import jax.numpy as jnp
from jax import lax
from jax.experimental import pallas as pl
from jax.experimental.pallas import tpu as pltpu
from jax.experimental.pallas import tpu_sc as plsc

N = 10000
NP = 10240
D = 128
E = 320000
NC, NS = 2, 16
NW = NC * NS
EP = NW * NP // 1
E_PER_T = 10240
CHUNK = 80
NCHUNK = E_PER_T // CHUNK
NCH = NCHUNK // 2
SHIFT = 14
NBUF = 4
RPT = NP // NS
BLK = 2048
NBLK = NP // BLK

_mesh = plsc.VectorSubcoreMesh(core_axis_name="c", subcore_axis_name="s")


def _degree_body(pk_hbm, zrow_hbm, hist_hbm, maxp_hbm,
                 pk_v, hist_v, max_v):
    c = lax.axis_index("c")
    s = lax.axis_index("s")
    wid = s * NC + c
    pltpu.sync_copy(pk_hbm.at[pl.ds(wid * E_PER_T, E_PER_T)], pk_v)
    pltpu.sync_copy(zrow_hbm, hist_v)
    max_v[...] = jnp.zeros((16,), jnp.int32)
    ones = jnp.full((16,), 1.0, jnp.float32)
    zero16 = jnp.zeros((16,), jnp.int32)
    mask = jnp.full((16,), (1 << SHIFT) - 1, jnp.int32)

    def body(i, carry):
        pk = pk_v[pl.ds(i * 16, 16)]
        s16 = lax.shift_right_logical(pk, SHIFT)
        t16 = lax.bitwise_and(pk, mask)
        plsc.addupdate_scatter(hist_v, [s16], ones)
        m16 = jnp.maximum(jnp.where(s16 < N, s16, zero16),
                          jnp.where(t16 < N, t16, zero16))
        max_v[...] = jnp.maximum(max_v[...], m16)
        return carry

    lax.fori_loop(0, E_PER_T // 16, body, 0)
    pltpu.sync_copy(hist_v, hist_hbm.at[wid])
    pltpu.sync_copy(max_v, maxp_hbm.at[wid])


@jax.jit
def _degree_call(packed, zrow):
    return pl.kernel(
        _degree_body,
        out_type=(
            jax.ShapeDtypeStruct((NW, NP), jnp.float32),
            jax.ShapeDtypeStruct((NW, 16), jnp.int32),
        ),
        mesh=_mesh,
        scratch_types=[
            pltpu.VMEM((E_PER_T,), jnp.int32),
            pltpu.VMEM((NP,), jnp.float32),
            pltpu.VMEM((16,), jnp.int32),
        ],
        compiler_params=pltpu.CompilerParams(
            needs_layout_passes=False, use_tc_tiling_on_sc=False),
    )(packed, zrow)


def _dis_body(maxp_ref, hist_ref, dis_ref):
    nn = jnp.max(maxp_ref[...]) + 1
    cnt = jnp.sum(hist_ref[...], axis=0)
    r = lax.broadcasted_iota(jnp.int32, (NP // 128, 128), 0)
    l = lax.broadcasted_iota(jnp.int32, (NP // 128, 128), 1)
    node = r * 128 + l
    deg = cnt + (node < nn).astype(jnp.float32)
    dis_ref[...] = jnp.where(deg > 0.0, lax.rsqrt(deg), 0.0)


@jax.jit
def _dis_call(maxp, hists4):
    return pl.pallas_call(
        _dis_body,
        in_specs=[
            pl.BlockSpec((NW, 16), lambda: (0, 0)),
            pl.BlockSpec((NW, NP // 128, 128), lambda: (0, 0, 0)),
        ],
        out_specs=pl.BlockSpec((NP // 128, 128), lambda: (0, 0)),
        out_shape=jax.ShapeDtypeStruct((NP // 128, 128), jnp.float32),
    )(maxp, hists4)


def _matmul_body(x_ref, wt_ref, h_ref):
    h_ref[...] = jnp.dot(x_ref[...], wt_ref[...],
                         preferred_element_type=jnp.float32)


@jax.jit
def _matmul_call(x, wt):
    return pl.pallas_call(
        _matmul_body,
        grid=(NBLK,),
        in_specs=[
            pl.BlockSpec((BLK, D), lambda i: (i, 0)),
            pl.BlockSpec((D, D), lambda i: (0, 0)),
        ],
        out_specs=pl.BlockSpec((BLK, D), lambda i: (i, 0)),
        out_shape=jax.ShapeDtypeStruct((NP, D), jnp.float32),
    )(x, wt)


def _scale_body(h_ref, dis_ref, g_ref):
    g_ref[...] = dis_ref[...] * h_ref[...]


@jax.jit
def _scale_call(h, dis_col):
    return pl.pallas_call(
        _scale_body,
        grid=(NBLK,),
        in_specs=[
            pl.BlockSpec((BLK, D), lambda i: (i, 0)),
            pl.BlockSpec((BLK, 1), lambda i: (i, 0)),
        ],
        out_specs=pl.BlockSpec((BLK, D), lambda i: (i, 0)),
        out_shape=jax.ShapeDtypeStruct((NP, D), jnp.float32),
    )(h, dis_col)


def _scatter_body(g_hbm, pk_hbm, zblk_hbm, part_hbm,
                  pk_v, srcb, tgtb, rows, gsems, ssems, acc):
    c = lax.axis_index("c")
    s = lax.axis_index("s")
    wid = c * NS + s
    pltpu.sync_copy(pk_hbm.at[pl.ds(wid * E_PER_T, NCH * CHUNK)], pk_v)
    pltpu.sync_copy(zblk_hbm, acc.at[pl.ds(s * RPT, RPT)])
    mask = jnp.full((16,), (1 << SHIFT) - 1, jnp.int32)

    def unpack(j, k):
        for i in range(CHUNK // 16):
            pk = pk_v[pl.ds(j * CHUNK + i * 16, 16)]
            srcb[k][pl.ds(i * 16, 16)] = lax.shift_right_logical(pk, SHIFT)
            tgtb[k][pl.ds(i * 16, 16)] = lax.bitwise_and(pk, mask)

    plsc.subcore_barrier()

    def run_half():
        for k in range(NBUF):
            unpack(k, k)
            pltpu.async_copy(g_hbm.at[srcb[k]], rows[k], gsems[k])

        def body(jj, carry):
            j = jj * NBUF
            for k in range(NBUF):
                pltpu.make_async_copy(
                    g_hbm.at[srcb[k]], rows[k], gsems[k]).wait()
                pltpu.async_copy(
                    rows[k], acc.at[tgtb[k]], ssems[k], add=True)
            for k in range(NBUF):
                pltpu.make_async_copy(
                    rows[k], acc.at[tgtb[k]], ssems[k]).wait()

                @pl.when(j + NBUF + k < NCH)
                def _():
                    unpack(j + NBUF + k, k)
                    pltpu.async_copy(g_hbm.at[srcb[k]], rows[k], gsems[k])
            return carry

        lax.fori_loop(0, NCH // NBUF, body, 0)

    run_half()
    pltpu.sync_copy(
        pk_hbm.at[pl.ds(wid * E_PER_T + NCH * CHUNK, NCH * CHUNK)], pk_v)
    run_half()
    plsc.subcore_barrier()
    pltpu.sync_copy(acc.at[pl.ds(s * RPT, RPT)],
                    part_hbm.at[pl.ds(c * NP + s * RPT, RPT)])


@jax.jit
def _scatter_call(g, packed, zblk):
    return pl.kernel(
        _scatter_body,
        out_type=jax.ShapeDtypeStruct((NC * NP, D), jnp.float32),
        mesh=_mesh,
        scratch_types=[
            pltpu.VMEM((NCH * CHUNK,), jnp.int32),
            [pltpu.VMEM((CHUNK,), jnp.int32) for _ in range(NBUF)],
            [pltpu.VMEM((CHUNK,), jnp.int32) for _ in range(NBUF)],
            [pltpu.VMEM((CHUNK, D), jnp.float32) for _ in range(NBUF)],
            [pltpu.SemaphoreType.DMA for _ in range(NBUF)],
            [pltpu.SemaphoreType.DMA for _ in range(NBUF)],
            pltpu.VMEM_SHARED((NP, D), jnp.float32),
        ],
        compiler_params=pltpu.CompilerParams(
            needs_layout_passes=False, use_tc_tiling_on_sc=False),
    )(g, packed, zblk)


def _out_body(p_ref, g_ref, dis_ref, o_ref):
    o_ref[...] = dis_ref[...] * (p_ref[0] + p_ref[1] + g_ref[...])


OBLK = 2000


@jax.jit
def _out_call(parts3, g, dis_col):
    return pl.pallas_call(
        _out_body,
        grid=(N // OBLK,),
        in_specs=[
            pl.BlockSpec((NC, OBLK, D), lambda i: (0, i, 0)),
            pl.BlockSpec((OBLK, D), lambda i: (i, 0)),
            pl.BlockSpec((OBLK, 1), lambda i: (i, 0)),
        ],
        out_specs=pl.BlockSpec((OBLK, D), lambda i: (i, 0)),
        out_shape=jax.ShapeDtypeStruct((N, D), jnp.float32),
    )(parts3, g, dis_col)


def kernel(x, edge_index, W):
    edges = edge_index.astype(jnp.int32)
    dummy = N + (jnp.arange(NW * E_PER_T - E, dtype=jnp.int32) % (NP - N))
    packed = jnp.concatenate([(edges[0] << SHIFT) | edges[1],
                              (dummy << SHIFT) | dummy])
    x_pad = jnp.pad(x, ((0, NP - N), (0, 0)))
    wt = W.T

    zrow = jnp.zeros((NP,), jnp.float32)
    hists, maxp = _degree_call(packed, zrow)
    h = _matmul_call(x_pad, wt)

    dis_flat = _dis_call(maxp, hists.reshape(NW, NP // 128, 128))
    dis_col = dis_flat.reshape(NP, 1)

    g = _scale_call(h, dis_col)

    zblk = jnp.zeros((RPT, D), jnp.float32)
    parts = _scatter_call(g, packed, zblk)

    return _out_call(parts.reshape(NC, NP, D), g, dis_col)


# speedup vs baseline: 1.0056x; 1.0056x over previous
import functools

import jax
import jax.numpy as jnp
from jax import lax
from jax.experimental import pallas as pl
from jax.experimental.pallas import tpu as pltpu
from jax.experimental.pallas import tpu_sc as plsc

N = 10000
NP = 10240
D = 128
E = 320000
NC, NS = 2, 16
NW = NC * NS
EP = NW * NP // 1
E_PER_T = 10240
CHUNK = 64
NCHUNK = E_PER_T // CHUNK
NCH = NCHUNK
SHIFT = 14
NBUF = 4
RPT = NP // NS
BLK = 2048
NBLK = NP // BLK

_mesh = plsc.VectorSubcoreMesh(core_axis_name="c", subcore_axis_name="s")


def _degree_body(pk_hbm, zrow_hbm, hist_hbm, maxp_hbm,
                 pk_v, hist_v, max_v):
    c = lax.axis_index("c")
    s = lax.axis_index("s")
    wid = s * NC + c
    pltpu.sync_copy(pk_hbm.at[pl.ds(wid * E_PER_T, E_PER_T)], pk_v)
    pltpu.sync_copy(zrow_hbm, hist_v)
    max_v[...] = jnp.zeros((16,), jnp.int32)
    ones = jnp.full((16,), 1.0, jnp.float32)
    zero16 = jnp.zeros((16,), jnp.int32)
    mask = jnp.full((16,), (1 << SHIFT) - 1, jnp.int32)

    def body(i, carry):
        pk = pk_v[pl.ds(i * 16, 16)]
        s16 = lax.shift_right_logical(pk, SHIFT)
        t16 = lax.bitwise_and(pk, mask)
        plsc.addupdate_scatter(hist_v, [s16], ones)
        m16 = jnp.maximum(jnp.where(s16 < N, s16, zero16),
                          jnp.where(t16 < N, t16, zero16))
        max_v[...] = jnp.maximum(max_v[...], m16)
        return carry

    lax.fori_loop(0, E_PER_T // 16, body, 0)
    pltpu.sync_copy(hist_v, hist_hbm.at[wid])
    pltpu.sync_copy(max_v, maxp_hbm.at[wid])


@jax.jit
def _degree_call(packed, zrow):
    return pl.kernel(
        _degree_body,
        out_type=(
            jax.ShapeDtypeStruct((NW, NP), jnp.float32),
            jax.ShapeDtypeStruct((NW, 16), jnp.int32),
        ),
        mesh=_mesh,
        scratch_types=[
            pltpu.VMEM((E_PER_T,), jnp.int32),
            pltpu.VMEM((NP,), jnp.float32),
            pltpu.VMEM((16,), jnp.int32),
        ],
        compiler_params=pltpu.CompilerParams(
            needs_layout_passes=False, use_tc_tiling_on_sc=False),
    )(packed, zrow)


def _dis_body(maxp_ref, hist_ref, dis_ref):
    nn = jnp.max(maxp_ref[...]) + 1
    cnt = jnp.sum(hist_ref[...], axis=0)
    r = lax.broadcasted_iota(jnp.int32, (NP // 128, 128), 0)
    l = lax.broadcasted_iota(jnp.int32, (NP // 128, 128), 1)
    node = r * 128 + l
    deg = cnt + (node < nn).astype(jnp.float32)
    dis_ref[...] = jnp.where(deg > 0.0, lax.rsqrt(deg), 0.0)


@jax.jit
def _dis_call(maxp, hists4):
    return pl.pallas_call(
        _dis_body,
        in_specs=[
            pl.BlockSpec((NW, 16), lambda: (0, 0)),
            pl.BlockSpec((NW, NP // 128, 128), lambda: (0, 0, 0)),
        ],
        out_specs=pl.BlockSpec((NP // 128, 128), lambda: (0, 0)),
        out_shape=jax.ShapeDtypeStruct((NP // 128, 128), jnp.float32),
    )(maxp, hists4)


def _matmul_body(x_ref, wt_ref, h_ref):
    h_ref[...] = jnp.dot(x_ref[...], wt_ref[...],
                         preferred_element_type=jnp.float32)


@jax.jit
def _matmul_call(x, wt):
    return pl.pallas_call(
        _matmul_body,
        grid=(NBLK,),
        in_specs=[
            pl.BlockSpec((BLK, D), lambda i: (i, 0)),
            pl.BlockSpec((D, D), lambda i: (0, 0)),
        ],
        out_specs=pl.BlockSpec((BLK, D), lambda i: (i, 0)),
        out_shape=jax.ShapeDtypeStruct((NP, D), jnp.float32),
    )(x, wt)


def _scale_body(h_ref, dis_ref, g_ref):
    g_ref[...] = dis_ref[...] * h_ref[...]


@jax.jit
def _scale_call(h, dis_col):
    return pl.pallas_call(
        _scale_body,
        grid=(NBLK,),
        in_specs=[
            pl.BlockSpec((BLK, D), lambda i: (i, 0)),
            pl.BlockSpec((BLK, 1), lambda i: (i, 0)),
        ],
        out_specs=pl.BlockSpec((BLK, D), lambda i: (i, 0)),
        out_shape=jax.ShapeDtypeStruct((NP, D), jnp.float32),
    )(h, dis_col)


def _scatter_body(g_hbm, pk_hbm, zblk_hbm, part_hbm,
                  pk_v, srcb, tgtb, rows, gsems, ssems, acc):
    c = lax.axis_index("c")
    s = lax.axis_index("s")
    wid = c * NS + s
    pltpu.sync_copy(pk_hbm.at[pl.ds(wid * E_PER_T, NCH * CHUNK)], pk_v)
    pltpu.sync_copy(zblk_hbm, acc.at[pl.ds(s * RPT, RPT)])
    mask = jnp.full((16,), (1 << SHIFT) - 1, jnp.int32)

    def unpack(j, k):
        for i in range(CHUNK // 16):
            pk = pk_v[pl.ds(j * CHUNK + i * 16, 16)]
            srcb[k][pl.ds(i * 16, 16)] = lax.shift_right_logical(pk, SHIFT)
            tgtb[k][pl.ds(i * 16, 16)] = lax.bitwise_and(pk, mask)

    plsc.subcore_barrier()

    def run_half():
        for k in range(NBUF):
            unpack(k, k)
            pltpu.async_copy(g_hbm.at[srcb[k]], rows[k], gsems[k])

        def body(jj, carry):
            j = jj * NBUF
            for k in range(NBUF):
                pltpu.make_async_copy(
                    g_hbm.at[srcb[k]], rows[k], gsems[k]).wait()
                pltpu.async_copy(
                    rows[k], acc.at[tgtb[k]], ssems[k], add=True)
            for k in range(NBUF):
                pltpu.make_async_copy(
                    rows[k], acc.at[tgtb[k]], ssems[k]).wait()

                @pl.when(j + NBUF + k < NCH)
                def _():
                    unpack(j + NBUF + k, k)
                    pltpu.async_copy(g_hbm.at[srcb[k]], rows[k], gsems[k])
            return carry

        lax.fori_loop(0, NCH // NBUF, body, 0)

    run_half()
    plsc.subcore_barrier()
    pltpu.sync_copy(acc.at[pl.ds(s * RPT, RPT)],
                    part_hbm.at[pl.ds(c * NP + s * RPT, RPT)])


@jax.jit
def _scatter_call(g, packed, zblk):
    return pl.kernel(
        _scatter_body,
        out_type=jax.ShapeDtypeStruct((NC * NP, D), jnp.float32),
        mesh=_mesh,
        scratch_types=[
            pltpu.VMEM((NCH * CHUNK,), jnp.int32),
            [pltpu.VMEM((CHUNK,), jnp.int32) for _ in range(NBUF)],
            [pltpu.VMEM((CHUNK,), jnp.int32) for _ in range(NBUF)],
            [pltpu.VMEM((CHUNK, D), jnp.float32) for _ in range(NBUF)],
            [pltpu.SemaphoreType.DMA for _ in range(NBUF)],
            [pltpu.SemaphoreType.DMA for _ in range(NBUF)],
            pltpu.VMEM_SHARED((NP, D), jnp.float32),
        ],
        compiler_params=pltpu.CompilerParams(
            needs_layout_passes=False, use_tc_tiling_on_sc=False),
    )(g, packed, zblk)


def _out_body(p_ref, g_ref, dis_ref, o_ref):
    o_ref[...] = dis_ref[...] * (p_ref[0] + p_ref[1] + g_ref[...])


OBLK = 2000


@jax.jit
def _out_call(parts3, g, dis_col):
    return pl.pallas_call(
        _out_body,
        grid=(N // OBLK,),
        in_specs=[
            pl.BlockSpec((NC, OBLK, D), lambda i: (0, i, 0)),
            pl.BlockSpec((OBLK, D), lambda i: (i, 0)),
            pl.BlockSpec((OBLK, 1), lambda i: (i, 0)),
        ],
        out_specs=pl.BlockSpec((OBLK, D), lambda i: (i, 0)),
        out_shape=jax.ShapeDtypeStruct((N, D), jnp.float32),
    )(parts3, g, dis_col)


def kernel(x, edge_index, W):
    edges = edge_index.astype(jnp.int32)
    dummy = N + (jnp.arange(NW * E_PER_T - E, dtype=jnp.int32) % (NP - N))
    packed = jnp.concatenate([(edges[0] << SHIFT) | edges[1],
                              (dummy << SHIFT) | dummy])
    x_pad = jnp.pad(x, ((0, NP - N), (0, 0)))
    wt = W.T

    zrow = jnp.zeros((NP,), jnp.float32)
    hists, maxp = _degree_call(packed, zrow)
    h = _matmul_call(x_pad, wt)

    dis_flat = _dis_call(maxp, hists.reshape(NW, NP // 128, 128))
    dis_col = dis_flat.reshape(NP, 1)

    g = _scale_call(h, dis_col)

    zblk = jnp.zeros((RPT, D), jnp.float32)
    parts = _scatter_call(g, packed, zblk)

    return _out_call(parts.reshape(NC, NP, D), g, dis_col)
